# baseline (speedup 1.0000x reference)
import functools

import jax
import jax.numpy as jnp
from jax import lax
from jax.experimental import pallas as pl
from jax.experimental.pallas import tpu as pltpu
from jax.experimental.pallas import tpu_sc as plsc

N_NODES = 10000
N_EDGES = 320000
D_FEAT = 128
D_HID = 16
N_CLASSES = 40
D2 = 48

NC = 2
NS = 16
NW = NC * NS
NPAD = 10240
RPT = NPAD // NS
EPW = N_EDGES // NW
CHUNK = 80
NCHUNK = EPW // CHUNK
NBUF = 10
LOOK = 5

f32 = jnp.float32
i32 = jnp.int32

_MESH = plsc.VectorSubcoreMesh(core_axis_name="c", subcore_axis_name="s",
                               num_cores=NC, num_subcores=NS)
_SC_PARAMS = pltpu.CompilerParams(use_tc_tiling_on_sc=False,
                                  needs_layout_passes=False)


def _deg_body(cidx_hbm, outa_hbm, outb_hbm, cidx_v, ones_v, acc, sem):
    cid = lax.axis_index("c")
    sid = lax.axis_index("s")
    wid = cid * NS + sid

    pltpu.sync_copy(cidx_hbm.at[wid], cidx_v)

    def fill(i, _):
        ones_v[pl.ds(i * 16, 16)] = jnp.full((16,), 1.0, dtype=f32)
        return 0
    lax.fori_loop(0, RPT // 16, fill, 0)

    sl = pl.ds(sid * RPT, RPT)
    pltpu.sync_copy(ones_v, acc.at[sl])
    plsc.subcore_barrier()

    def step(j, _):
        pltpu.async_copy(ones_v.at[pl.ds(0, CHUNK)], acc.at[cidx_v.at[j]],
                         sem, add=True)
        return 0
    lax.fori_loop(0, NCHUNK, step, 0)

    def drain(j, _):
        pltpu.make_async_copy(ones_v.at[pl.ds(0, CHUNK)],
                              acc.at[cidx_v.at[j]], sem).wait()
        return 0
    lax.fori_loop(0, NCHUNK, drain, 0)

    plsc.subcore_barrier()

    @pl.when(cid == 0)
    def _():
        pltpu.sync_copy(acc.at[sl], outa_hbm.at[sl])

    @pl.when(cid == 1)
    def _():
        pltpu.sync_copy(acc.at[sl], outb_hbm.at[sl])


_deg_kernel = functools.partial(
    pl.kernel,
    out_type=[jax.ShapeDtypeStruct((NPAD,), f32),
              jax.ShapeDtypeStruct((NPAD,), f32)],
    mesh=_MESH,
    scratch_types=[
        pltpu.VMEM((NCHUNK, CHUNK), i32),
        pltpu.VMEM((RPT,), f32),
        pltpu.VMEM_SHARED((NPAD,), f32),
        pltpu.SemaphoreType.DMA,
    ],
    compiler_params=_SC_PARAMS,
)(_deg_body)


def _edge_body(d, h_hbm, dega_hbm, degb_hbm, ridx_hbm, cidx_hbm,
               sa_hbm, sb_hbm, td_hbm,
               ridx_v, cidx_v, rows, hv, dv, da_v, db_v, acc, *sems):
    cid = lax.axis_index("c")
    sid = lax.axis_index("s")
    wid = cid * NS + sid
    sl = pl.ds(sid * RPT, RPT)
    rv = d // 16

    pltpu.sync_copy(ridx_hbm.at[wid], ridx_v)
    pltpu.sync_copy(cidx_hbm.at[wid], cidx_v)
    pltpu.sync_copy(dega_hbm.at[sl], da_v)
    pltpu.sync_copy(degb_hbm.at[sl], db_v)
    pltpu.sync_copy(h_hbm.at[sl, pl.ds(0, d)], hv)

    def newton(g, _):
        gsl = pl.ds(g * 16, 16)
        deg = da_v[gsl] + db_v[gsl] - 1.0
        y = plsc.bitcast(
            0x5F3759DF - (plsc.bitcast(deg, i32) >> 1), f32)
        hx = 0.5 * deg
        y = y * (1.5 - hx * y * y)
        y = y * (1.5 - hx * y * y)
        y = y * (1.5 - hx * y * y)
        dv[gsl] = y
        return 0
    lax.fori_loop(0, RPT // 16, newton, 0)

    def scale(g, _):
        d16 = dv[pl.ds(g * 16, 16)]
        for k in range(16):
            r = g * 16 + k
            bc = jnp.full((16,), d16[k], dtype=f32)
            for v in range(rv):
                csl = pl.ds(v * 16, 16)
                hv[r, csl] = hv[r, csl] * bc
        return 0
    lax.fori_loop(0, RPT // 16, scale, 0)

    pltpu.sync_copy(hv, td_hbm.at[sl])

    def halve(g, _):
        for k in range(16):
            r = g * 16 + k
            for v in range(rv):
                csl = pl.ds(v * 16, 16)
                hv[r, csl] = hv[r, csl] * 0.5
        return 0
    lax.fori_loop(0, RPT // 16, halve, 0)

    pltpu.sync_copy(hv, acc.at[sl])
    plsc.subcore_barrier()

    gsems = sems[:NBUF]
    ssems = sems[NBUF:]
    for b in range(LOOK):
        pltpu.async_copy(td_hbm.at[ridx_v.at[b]], rows.at[b], gsems[b])

    def group(g, _):
        for b in range(NBUF):
            j = g * NBUF + b
            jn = j + LOOK
            bn = (b + LOOK) % NBUF

            @pl.when(j < NCHUNK)
            def _():
                pltpu.make_async_copy(td_hbm.at[ridx_v.at[j]], rows.at[b],
                                      gsems[b]).wait()
                pltpu.async_copy(rows.at[b], acc.at[cidx_v.at[j]], ssems[b],
                                 add=True)

                @pl.when(jn < NCHUNK)
                def _():
                    @pl.when(jn >= NBUF)
                    def _():
                        pltpu.make_async_copy(rows.at[bn],
                                              acc.at[cidx_v.at[j]],
                                              ssems[bn]).wait()
                    pltpu.async_copy(td_hbm.at[ridx_v.at[jn]], rows.at[bn],
                                     gsems[bn])
        return 0
    lax.fori_loop(0, (NCHUNK + NBUF - 1) // NBUF, group, 0)

    for b in range(NBUF):
        pltpu.make_async_copy(rows.at[b], acc.at[cidx_v.at[b]],
                              ssems[b]).wait()
    plsc.subcore_barrier()

    pltpu.sync_copy(acc.at[sl], hv)

    def scale_out(g, _):
        d16 = dv[pl.ds(g * 16, 16)]
        for k in range(16):
            r = g * 16 + k
            bc = jnp.full((16,), d16[k], dtype=f32)
            for v in range(rv):
                csl = pl.ds(v * 16, 16)
                hv[r, csl] = hv[r, csl] * bc
        return 0
    lax.fori_loop(0, RPT // 16, scale_out, 0)

    @pl.when(cid == 0)
    def _():
        pltpu.sync_copy(hv, sa_hbm.at[sl, pl.ds(0, d)])

    @pl.when(cid == 1)
    def _():
        pltpu.sync_copy(hv, sb_hbm.at[sl, pl.ds(0, d)])


def _make_edge_kernel(d):
    return functools.partial(
        pl.kernel,
        out_type=[jax.ShapeDtypeStruct((NPAD, 128), f32),
                  jax.ShapeDtypeStruct((NPAD, 128), f32),
                  jax.ShapeDtypeStruct((NPAD, d), f32)],
        mesh=_MESH,
        scratch_types=[
            pltpu.VMEM((NCHUNK, CHUNK), i32),
            pltpu.VMEM((NCHUNK, CHUNK), i32),
            pltpu.VMEM((NBUF, CHUNK, d), f32),
            pltpu.VMEM((RPT, d), f32),
            pltpu.VMEM((RPT,), f32),
            pltpu.VMEM((RPT,), f32),
            pltpu.VMEM((RPT,), f32),
            pltpu.VMEM_SHARED((NPAD, d), f32),
        ] + [pltpu.SemaphoreType.DMA] * (2 * NBUF),
        compiler_params=_SC_PARAMS,
    )(functools.partial(_edge_body, d))


_edge_kernel_h = _make_edge_kernel(D_HID)
_edge_kernel_c = _make_edge_kernel(D2)


_RB = 1024
_GRID = (NPAD // _RB,)


def _tc1_body(x_ref, w1_ref, h1_ref):
    h1_ref[...] = jnp.dot(x_ref[...], w1_ref[...], preferred_element_type=f32)


def _tc2_body(sa_ref, sb_ref, w2_ref, b1_ref, h2_ref):
    u = sa_ref[:, :D_HID] + sb_ref[:, :D_HID]
    h = jnp.maximum(u + b1_ref[...], 0.0)
    h2_ref[...] = jnp.dot(h, w2_ref[...], preferred_element_type=f32)


def _tc3_body(sa_ref, sb_ref, b2_ref, out_ref):
    z = (sa_ref[:, :N_CLASSES] + sb_ref[:, :N_CLASSES]) + b2_ref[...]
    m = jnp.max(z, axis=1, keepdims=True)
    lse = jnp.log(jnp.sum(jnp.exp(z - m), axis=1, keepdims=True)) + m
    out_ref[...] = z - lse


def _blk128():
    return pl.BlockSpec((_RB, 128), lambda i: (i, 0))


def _full_spec(a, b):
    return pl.BlockSpec((a, b), lambda i: (0, 0))


def kernel(x, edge_index, W1, b1, W2, b2):
    ei = edge_index.astype(i32)
    ridx = ei[0].reshape(NW, NCHUNK, CHUNK)
    cidx = ei[1].reshape(NW, NCHUNK, CHUNK)
    xp = jnp.pad(x, ((0, NPAD - N_NODES), (0, 0)))
    w1p = jnp.pad(W1, ((0, 0), (0, 128 - D_HID)))
    w2p = jnp.pad(W2, ((0, 0), (0, 128 - N_CLASSES)))
    b1r = b1.reshape(1, D_HID)
    b2r = b2.reshape(1, N_CLASSES)

    dega, degb = _deg_kernel(cidx)

    h1 = pl.pallas_call(
        _tc1_body,
        grid=_GRID,
        in_specs=[pl.BlockSpec((_RB, D_FEAT), lambda i: (i, 0)),
                  _full_spec(D_FEAT, 128)],
        out_specs=_blk128(),
        out_shape=jax.ShapeDtypeStruct((NPAD, 128), f32),
    )(xp, w1p)

    s1a, s1b, _ = _edge_kernel_h(h1, dega, degb, ridx, cidx)

    h2 = pl.pallas_call(
        _tc2_body,
        grid=_GRID,
        in_specs=[_blk128(), _blk128(), _full_spec(D_HID, 128),
                  _full_spec(1, D_HID)],
        out_specs=_blk128(),
        out_shape=jax.ShapeDtypeStruct((NPAD, 128), f32),
    )(s1a, s1b, w2p, b1r)

    s2a, s2b, _ = _edge_kernel_c(h2, dega, degb, ridx, cidx)

    out = pl.pallas_call(
        _tc3_body,
        grid=_GRID,
        in_specs=[_blk128(), _blk128(), _full_spec(1, N_CLASSES)],
        out_specs=pl.BlockSpec((_RB, N_CLASSES), lambda i: (i, 0)),
        out_shape=jax.ShapeDtypeStruct((NPAD, N_CLASSES), f32),
    )(s2a, s2b, b2r)

    return out[:N_NODES]


# speedup vs baseline: 1.0006x; 1.0006x over previous
import functools

import jax
import jax.numpy as jnp
from jax import lax
from jax.experimental import pallas as pl
from jax.experimental.pallas import tpu as pltpu
from jax.experimental.pallas import tpu_sc as plsc

N_NODES = 10000
N_EDGES = 320000
D_FEAT = 128
D_HID = 16
N_CLASSES = 40
D2 = 48

NC = 2
NS = 16
NW = NC * NS
NPAD = 10240
RPT = NPAD // NS
EPW = N_EDGES // NW
CHUNK = 80
NCHUNK = EPW // CHUNK
NBUF = 5

f32 = jnp.float32
i32 = jnp.int32

_MESH = plsc.VectorSubcoreMesh(core_axis_name="c", subcore_axis_name="s",
                               num_cores=NC, num_subcores=NS)
_SC_PARAMS = pltpu.CompilerParams(use_tc_tiling_on_sc=False,
                                  needs_layout_passes=False)


def _deg_body(cidx_hbm, outa_hbm, outb_hbm, cidx_v, ones_v, acc, sem):
    cid = lax.axis_index("c")
    sid = lax.axis_index("s")
    wid = cid * NS + sid

    pltpu.sync_copy(cidx_hbm.at[wid], cidx_v)

    def fill(i, _):
        ones_v[pl.ds(i * 16, 16)] = jnp.full((16,), 1.0, dtype=f32)
        return 0
    lax.fori_loop(0, RPT // 16, fill, 0)

    sl = pl.ds(sid * RPT, RPT)
    pltpu.sync_copy(ones_v, acc.at[sl])
    plsc.subcore_barrier()

    def step(j, _):
        pltpu.async_copy(ones_v.at[pl.ds(0, CHUNK)], acc.at[cidx_v.at[j]],
                         sem, add=True)
        return 0
    lax.fori_loop(0, NCHUNK, step, 0)

    def drain(j, _):
        pltpu.make_async_copy(ones_v.at[pl.ds(0, CHUNK)],
                              acc.at[cidx_v.at[j]], sem).wait()
        return 0
    lax.fori_loop(0, NCHUNK, drain, 0)

    plsc.subcore_barrier()

    @pl.when(cid == 0)
    def _():
        pltpu.sync_copy(acc.at[sl], outa_hbm.at[sl])

    @pl.when(cid == 1)
    def _():
        pltpu.sync_copy(acc.at[sl], outb_hbm.at[sl])


_deg_kernel = functools.partial(
    pl.kernel,
    out_type=[jax.ShapeDtypeStruct((NPAD,), f32),
              jax.ShapeDtypeStruct((NPAD,), f32)],
    mesh=_MESH,
    scratch_types=[
        pltpu.VMEM((NCHUNK, CHUNK), i32),
        pltpu.VMEM((RPT,), f32),
        pltpu.VMEM_SHARED((NPAD,), f32),
        pltpu.SemaphoreType.DMA,
    ],
    compiler_params=_SC_PARAMS,
)(_deg_body)


def _edge_body(d, h_hbm, dega_hbm, degb_hbm, ridx_hbm, cidx_hbm,
               sa_hbm, sb_hbm, td_hbm,
               ridx_v, cidx_v, rows, hv, dv, da_v, db_v, acc, *sems):
    cid = lax.axis_index("c")
    sid = lax.axis_index("s")
    wid = cid * NS + sid
    sl = pl.ds(sid * RPT, RPT)
    rv = d // 16

    pltpu.sync_copy(ridx_hbm.at[wid], ridx_v)
    pltpu.sync_copy(cidx_hbm.at[wid], cidx_v)
    pltpu.sync_copy(dega_hbm.at[sl], da_v)
    pltpu.sync_copy(degb_hbm.at[sl], db_v)
    pltpu.sync_copy(h_hbm.at[sl, pl.ds(0, d)], hv)

    def newton(g, _):
        gsl = pl.ds(g * 16, 16)
        deg = da_v[gsl] + db_v[gsl] - 1.0
        y = plsc.bitcast(
            0x5F3759DF - (plsc.bitcast(deg, i32) >> 1), f32)
        hx = 0.5 * deg
        y = y * (1.5 - hx * y * y)
        y = y * (1.5 - hx * y * y)
        y = y * (1.5 - hx * y * y)
        dv[gsl] = y
        return 0
    lax.fori_loop(0, RPT // 16, newton, 0)

    def scale(g, _):
        d16 = dv[pl.ds(g * 16, 16)]
        for k in range(16):
            r = g * 16 + k
            bc = jnp.full((16,), d16[k], dtype=f32)
            for v in range(rv):
                csl = pl.ds(v * 16, 16)
                hv[r, csl] = hv[r, csl] * bc
        return 0
    lax.fori_loop(0, RPT // 16, scale, 0)

    pltpu.sync_copy(hv, td_hbm.at[sl])

    def halve(g, _):
        for k in range(16):
            r = g * 16 + k
            for v in range(rv):
                csl = pl.ds(v * 16, 16)
                hv[r, csl] = hv[r, csl] * 0.5
        return 0
    lax.fori_loop(0, RPT // 16, halve, 0)

    pltpu.sync_copy(hv, acc.at[sl])
    plsc.subcore_barrier()

    for b in range(NBUF):
        pltpu.async_copy(td_hbm.at[ridx_v.at[b]], rows.at[b], sems[b])

    def group(g, _):
        for b in range(NBUF):
            j = g * NBUF + b
            pltpu.make_async_copy(td_hbm.at[ridx_v.at[j]], rows.at[b],
                                  sems[b]).wait()
            pltpu.sync_copy(rows.at[b], acc.at[cidx_v.at[j]], add=True)
            jn = j + NBUF

            @pl.when(jn < NCHUNK)
            def _():
                pltpu.async_copy(td_hbm.at[ridx_v.at[jn]], rows.at[b],
                                 sems[b])
        return 0
    lax.fori_loop(0, NCHUNK // NBUF, group, 0)

    plsc.subcore_barrier()

    pltpu.sync_copy(acc.at[sl], hv)

    def scale_out(g, _):
        d16 = dv[pl.ds(g * 16, 16)]
        for k in range(16):
            r = g * 16 + k
            bc = jnp.full((16,), d16[k], dtype=f32)
            for v in range(rv):
                csl = pl.ds(v * 16, 16)
                hv[r, csl] = hv[r, csl] * bc
        return 0
    lax.fori_loop(0, RPT // 16, scale_out, 0)

    @pl.when(cid == 0)
    def _():
        pltpu.sync_copy(hv, sa_hbm.at[sl, pl.ds(0, d)])

    @pl.when(cid == 1)
    def _():
        pltpu.sync_copy(hv, sb_hbm.at[sl, pl.ds(0, d)])


def _make_edge_kernel(d):
    return functools.partial(
        pl.kernel,
        out_type=[jax.ShapeDtypeStruct((NPAD, 128), f32),
                  jax.ShapeDtypeStruct((NPAD, 128), f32),
                  jax.ShapeDtypeStruct((NPAD, d), f32)],
        mesh=_MESH,
        scratch_types=[
            pltpu.VMEM((NCHUNK, CHUNK), i32),
            pltpu.VMEM((NCHUNK, CHUNK), i32),
            pltpu.VMEM((NBUF, CHUNK, d), f32),
            pltpu.VMEM((RPT, d), f32),
            pltpu.VMEM((RPT,), f32),
            pltpu.VMEM((RPT,), f32),
            pltpu.VMEM((RPT,), f32),
            pltpu.VMEM_SHARED((NPAD, d), f32),
        ] + [pltpu.SemaphoreType.DMA] * NBUF,
        compiler_params=_SC_PARAMS,
    )(functools.partial(_edge_body, d))


_edge_kernel_h = _make_edge_kernel(D_HID)
_edge_kernel_c = _make_edge_kernel(D2)


_RB = 1024
_GRID = (NPAD // _RB,)


def _tc1_body(x_ref, w1_ref, h1_ref):
    h1_ref[...] = jnp.dot(x_ref[...], w1_ref[...], preferred_element_type=f32)


def _tc2_body(sa_ref, sb_ref, w2_ref, b1_ref, h2_ref):
    u = sa_ref[:, :D_HID] + sb_ref[:, :D_HID]
    h = jnp.maximum(u + b1_ref[...], 0.0)
    h2_ref[...] = jnp.dot(h, w2_ref[...], preferred_element_type=f32)


def _tc3_body(sa_ref, sb_ref, b2_ref, out_ref):
    z = (sa_ref[:, :N_CLASSES] + sb_ref[:, :N_CLASSES]) + b2_ref[...]
    m = jnp.max(z, axis=1, keepdims=True)
    lse = jnp.log(jnp.sum(jnp.exp(z - m), axis=1, keepdims=True)) + m
    out_ref[...] = z - lse


def _blk128():
    return pl.BlockSpec((_RB, 128), lambda i: (i, 0))


def _full_spec(a, b):
    return pl.BlockSpec((a, b), lambda i: (0, 0))


def kernel(x, edge_index, W1, b1, W2, b2):
    ei = edge_index.astype(i32)
    ridx = ei[0].reshape(NW, NCHUNK, CHUNK)
    cidx = ei[1].reshape(NW, NCHUNK, CHUNK)
    xp = jnp.pad(x, ((0, NPAD - N_NODES), (0, 0)))
    w1p = jnp.pad(W1, ((0, 0), (0, 128 - D_HID)))
    w2p = jnp.pad(W2, ((0, 0), (0, 128 - N_CLASSES)))
    b1r = b1.reshape(1, D_HID)
    b2r = b2.reshape(1, N_CLASSES)

    dega, degb = _deg_kernel(cidx)

    h1 = pl.pallas_call(
        _tc1_body,
        grid=_GRID,
        in_specs=[pl.BlockSpec((_RB, D_FEAT), lambda i: (i, 0)),
                  _full_spec(D_FEAT, 128)],
        out_specs=_blk128(),
        out_shape=jax.ShapeDtypeStruct((NPAD, 128), f32),
    )(xp, w1p)

    s1a, s1b, _ = _edge_kernel_h(h1, dega, degb, ridx, cidx)

    h2 = pl.pallas_call(
        _tc2_body,
        grid=_GRID,
        in_specs=[_blk128(), _blk128(), _full_spec(D_HID, 128),
                  _full_spec(1, D_HID)],
        out_specs=_blk128(),
        out_shape=jax.ShapeDtypeStruct((NPAD, 128), f32),
    )(s1a, s1b, w2p, b1r)

    s2a, s2b, _ = _edge_kernel_c(h2, dega, degb, ridx, cidx)

    out = pl.pallas_call(
        _tc3_body,
        grid=_GRID,
        in_specs=[_blk128(), _blk128(), _full_spec(1, N_CLASSES)],
        out_specs=pl.BlockSpec((_RB, N_CLASSES), lambda i: (i, 0)),
        out_shape=jax.ShapeDtypeStruct((NPAD, N_CLASSES), f32),
    )(s2a, s2b, b2r)

    return out[:N_NODES]
